# baseline (speedup 1.0000x reference)
import functools

import jax
import jax.numpy as jnp
from jax import lax
from jax.experimental import pallas as pl
from jax.experimental.pallas import tpu as pltpu
from jax.experimental.pallas import tpu_sc as plsc

_EMBED = 128
_C = 128
_NBUF = 2
_GPB = 2


@functools.lru_cache(maxsize=None)
def _build(total_rows: int):
    info = plsc.get_sparse_core_info()
    nc, ns = info.num_cores, info.num_subcores
    nw = nc * ns
    b_per_w = total_rows // nw
    n_chunks = b_per_w // _C
    rows_per_buf = _GPB * _C
    n_super = n_chunks // _GPB
    n_outer = n_super // _NBUF
    mesh = plsc.VectorSubcoreMesh(core_axis_name="c", subcore_axis_name="s")

    @functools.partial(
        pl.kernel,
        mesh=mesh,
        out_type=jax.ShapeDtypeStruct((total_rows, _EMBED), jnp.float32),
        scratch_types=[
            pltpu.VMEM((n_chunks, _C), jnp.int32),
            pltpu.VMEM((_NBUF, rows_per_buf, _EMBED), jnp.float32),
            *([pltpu.SemaphoreType.DMA] * (2 * _NBUF)),
        ],
    )
    def gather_kernel(table_hbm, idx_hbm, out_hbm, idx_v, rows_v, *sems):
        gsem = sems[:_NBUF]
        ssem = sems[_NBUF:]
        wid = lax.axis_index("s") * nc + lax.axis_index("c")
        base = wid * b_per_w
        pltpu.sync_copy(idx_hbm.at[wid], idx_v)

        def gather(js, b, g):
            return pltpu.make_async_copy(
                table_hbm.at[idx_v.at[js * _GPB + g]],
                rows_v.at[b, pl.ds(g * _C, _C)],
                gsem[b])

        def scatter(js, b):
            return pltpu.make_async_copy(
                rows_v.at[b],
                out_hbm.at[pl.ds(base + js * rows_per_buf, rows_per_buf)],
                ssem[b])

        for b in range(_NBUF):
            for g in range(_GPB):
                gather(b, b, g).start()

        def body(o, carry):
            for b in range(_NBUF):
                js = o * _NBUF + b
                for g in range(_GPB):
                    gather(js, b, g).wait()
                scatter(js, b).start()

                @pl.when(o < n_outer - 1)
                def _(js=js, b=b):
                    scatter(js, b).wait()
                    for g in range(_GPB):
                        gather(js + _NBUF, b, g).start()

            return carry

        lax.fori_loop(0, n_outer, body, 0)
        for b in range(_NBUF):
            scatter(n_super - _NBUF + b, b).wait()

    def run(table, idx_flat):
        idx3 = idx_flat.reshape(nw, n_chunks, _C)
        return gather_kernel(table, idx3)

    return run


def kernel(indices, table):
    b, h = indices.shape
    total = b * h
    idx_flat = indices.reshape(-1).astype(jnp.int32)
    out = _build(total)(table, idx_flat)
    return out.reshape(b, h, _EMBED)


# speedup vs baseline: 1.0559x; 1.0559x over previous
import functools

import jax
import jax.numpy as jnp
from jax import lax
from jax.experimental import pallas as pl
from jax.experimental.pallas import tpu as pltpu
from jax.experimental.pallas import tpu_sc as plsc

_EMBED = 128
_C = 128
_NBUF = 4
_SBUF = 2


@functools.lru_cache(maxsize=None)
def _build(total_rows: int):
    info = plsc.get_sparse_core_info()
    nc, ns = info.num_cores, info.num_subcores
    nw = nc * ns
    b_per_w = total_rows // nw
    n_chunks = b_per_w // _C
    n_outer = n_chunks // _NBUF
    mesh = plsc.VectorSubcoreMesh(core_axis_name="c", subcore_axis_name="s")

    @functools.partial(
        pl.kernel,
        mesh=mesh,
        out_type=jax.ShapeDtypeStruct((total_rows, _EMBED), jnp.float32),
        scratch_types=[
            pltpu.VMEM((n_chunks, _C), jnp.int32),
            pltpu.VMEM((_NBUF, _C, _EMBED), jnp.float32),
            pltpu.VMEM_SHARED((ns, _SBUF, _C, _EMBED), jnp.float32),
            *([pltpu.SemaphoreType.DMA] * (_NBUF + 2 * _SBUF)),
        ],
    )
    def gather_kernel(table_hbm, idx_hbm, out_hbm, idx_v, rows_v, rows_sp,
                      *sems):
        gsem = sems[:_NBUF]
        csem = sems[_NBUF:_NBUF + _SBUF]
        ssem = sems[_NBUF + _SBUF:]
        sid = lax.axis_index("s")
        wid = sid * nc + lax.axis_index("c")
        base = wid * b_per_w
        pltpu.sync_copy(idx_hbm.at[wid], idx_v)

        def gather(j, b):
            return pltpu.make_async_copy(
                table_hbm.at[idx_v.at[j]], rows_v.at[b], gsem[b])

        def relay(b, sb):
            return pltpu.make_async_copy(
                rows_v.at[b], rows_sp.at[sid, sb], csem[sb])

        def scatter(j, sb):
            return pltpu.make_async_copy(
                rows_sp.at[sid, sb],
                out_hbm.at[pl.ds(base + j * _C, _C)],
                ssem[sb])

        for b in range(_NBUF):
            gather(b, b).start()

        def body(o, carry):
            for b in range(_NBUF):
                j = o * _NBUF + b
                sb = b % _SBUF
                gather(j, b).wait()

                @pl.when(j >= _SBUF)
                def _(j=j, sb=sb):
                    scatter(j - _SBUF, sb).wait()

                relay(b, sb).start()
                relay(b, sb).wait()
                scatter(j, sb).start()

                @pl.when(o < n_outer - 1)
                def _(j=j, b=b):
                    gather(j + _NBUF, b).start()

            return carry

        lax.fori_loop(0, n_outer, body, 0)
        for sb in range(_SBUF):
            scatter(n_chunks - _SBUF + sb, sb).wait()

    def run(table, idx_flat):
        idx3 = idx_flat.reshape(nw, n_chunks, _C)
        return gather_kernel(table, idx3)

    return run


def kernel(indices, table):
    b, h = indices.shape
    total = b * h
    idx_flat = indices.reshape(-1).astype(jnp.int32)
    out = _build(total)(table, idx_flat)
    return out.reshape(b, h, _EMBED)
